# Initial kernel scaffold; baseline (speedup 1.0000x reference)
import functools

import jax
import jax.numpy as jnp
import numpy as np
from jax import lax
from jax.experimental import pallas as pl
from jax.experimental.pallas import tpu as pltpu
from jax.experimental.pallas import tpu_sc as plsc

N = 10000
E = 320000
HID = 128
XW = 8
ROWW = HID + XW
NUM_G = 16

NC = 2
NS = 16
NW = NC * NS
EPW = E // NW
SUB = 80
NSUB = 5
CHUNK = SUB * NSUB
NCHUNK = EPW // CHUNK
IDXROWS = E // SUB
IDXR_PW = EPW // SUB
ROWS_PT = N // NS

_SC_MESH = plsc.VectorSubcoreMesh(core_axis_name="c", subcore_axis_name="s")


def _pre_body(h_ref, xp_ref, wa_ref, wb_ref, ax_ref, bx_ref):
    h = h_ref[...]
    xp = xp_ref[...]
    a = jnp.dot(h, wa_ref[...], preferred_element_type=jnp.float32)
    b = jnp.dot(h, wb_ref[...], preferred_element_type=jnp.float32)
    ax_ref[...] = jnp.concatenate([a, xp], axis=1)
    bx_ref[...] = jnp.concatenate([b, xp], axis=1)


def _tc_pre(h, xpad, We1a, We1b):
    return pl.pallas_call(
        _pre_body,
        out_shape=(jax.ShapeDtypeStruct((N, ROWW), jnp.float32),
                   jax.ShapeDtypeStruct((N, ROWW), jnp.float32)),
    )(h, xpad, We1a, We1b)


_EB = 2560
_EGRID = E // _EB
_G_OFFSETS = np.linspace(0.0, 10.0, NUM_G, dtype=np.float32)
_G_COEFF = float(-0.5 / (_G_OFFSETS[1] - _G_OFFSETS[0]) ** 2)


def _edge_body(ad_ref, bs_ref, ea_ref, wd_ref, wea_ref, be1_ref, we2_ref,
               be2_ref, winf_ref, binf_ref, wx1_ref, bx1_ref, wx2_ref,
               out_ref):
    ad = ad_ref[...]
    bs = bs_ref[...]
    t1pre = ad[:, :HID] + bs[:, :HID]
    rel = ad[:, HID:] - bs[:, HID:]
    d_sq = jnp.sum(rel * rel, axis=1, keepdims=True)
    dist = jnp.sqrt(d_sq + 1e-8)
    offs = jnp.asarray(_G_OFFSETS).reshape(1, NUM_G)
    dfeat = jnp.exp(_G_COEFF * (dist - offs) ** 2)
    t1 = (t1pre
          + jnp.dot(dfeat, wd_ref[...], preferred_element_type=jnp.float32)
          + be1_ref[...])
    ea = ea_ref[...]
    wea = wea_ref[...]
    for k in range(4):
        t1 = t1 + ea[:, k:k + 1] * wea[k:k + 1, :]
    u = t1 * jax.nn.sigmoid(t1)
    m1 = jnp.dot(u, we2_ref[...], preferred_element_type=jnp.float32) + be2_ref[...]
    mij = m1 * jax.nn.sigmoid(m1)
    eij = jax.nn.sigmoid(
        jnp.sum(mij * winf_ref[...], axis=1, keepdims=True) + binf_ref[...])
    v1 = jnp.dot(mij, wx1_ref[...], preferred_element_type=jnp.float32) + bx1_ref[...]
    v = v1 * jax.nn.sigmoid(v1)
    xg = jnp.tanh(jnp.sum(v * wx2_ref[...], axis=1, keepdims=True))
    xmsg = rel * (xg / (dist + 1.0))
    out_ref[...] = jnp.concatenate([mij * eij, xmsg], axis=1)


def _tc_edge(adx, bsx, edge_attr, We1d, We1e, be1, We2, be2, winf_row, binf,
             Wx1, bx1, wx2_row):
    full = lambda shape: pl.BlockSpec(shape, lambda i: (0, 0))
    return pl.pallas_call(
        _edge_body,
        grid=(_EGRID,),
        in_specs=[
            pl.BlockSpec((_EB, ROWW), lambda i: (i, 0)),
            pl.BlockSpec((_EB, ROWW), lambda i: (i, 0)),
            pl.BlockSpec((_EB, 4), lambda i: (i, 0)),
            full((NUM_G, HID)),
            full((4, HID)),
            full((1, HID)),
            full((HID, HID)),
            full((1, HID)),
            full((1, HID)),
            full((1, 1)),
            full((HID, HID)),
            full((1, HID)),
            full((1, HID)),
        ],
        out_specs=pl.BlockSpec((_EB, ROWW), lambda i: (i, 0)),
        out_shape=jax.ShapeDtypeStruct((E, ROWW), jnp.float32),
    )(adx, bsx, edge_attr, We1d, We1e, be1, We2, be2, winf_row, binf,
      Wx1, bx1, wx2_row)


def _node_body(h_ref, xp_ref, parts_ref, mask_ref, wn1a_ref, wn1b_ref,
               bn1_ref, wn2_ref, bn2_ref, hout_ref, xout_ref):
    h = h_ref[...]
    acc = parts_ref[0] + parts_ref[1]
    mi = acc[:, :HID]
    dx = acc[:, HID:]
    t1 = (jnp.dot(mi, wn1a_ref[...], preferred_element_type=jnp.float32)
          + jnp.dot(h, wn1b_ref[...], preferred_element_type=jnp.float32)
          + bn1_ref[...])
    t = t1 * jax.nn.sigmoid(t1)
    hout_ref[...] = h + jnp.dot(t, wn2_ref[...],
                                preferred_element_type=jnp.float32) + bn2_ref[...]
    xout_ref[...] = xp_ref[...] + dx * mask_ref[...]


def _tc_node(h, xpad, parts, mask_f, Wn1a, Wn1b, bn1, Wn2, bn2):
    return pl.pallas_call(
        _node_body,
        out_shape=(jax.ShapeDtypeStruct((N, HID), jnp.float32),
                   jax.ShapeDtypeStruct((N, XW), jnp.float32)),
    )(h, xpad, parts, mask_f, Wn1a, Wn1b, bn1, Wn2, bn2)


@functools.partial(
    pl.kernel,
    mesh=_SC_MESH,
    out_type=(jax.ShapeDtypeStruct((E, ROWW), jnp.float32),
              jax.ShapeDtypeStruct((E, ROWW), jnp.float32)),
    scratch_types=[
        pltpu.VMEM((NSUB, SUB), jnp.int32),
        pltpu.VMEM((NSUB, SUB), jnp.int32),
        pltpu.VMEM((CHUNK, ROWW), jnp.float32),
        pltpu.VMEM((CHUNK, ROWW), jnp.float32),
        pltpu.SemaphoreType.DMA,
    ],
)
def _sc_gather(ax_hbm, bx_hbm, dst2d_hbm, src2d_hbm, ad_out, bs_out,
               idxd, idxs, adb, bsb, sem):
    c = lax.axis_index("c")
    s = lax.axis_index("s")
    wid = s * NC + c
    row0 = wid * IDXR_PW
    e0w = wid * EPW

    def chunk(k, carry):
        r = row0 + k * NSUB
        pltpu.sync_copy(dst2d_hbm.at[pl.ds(r, NSUB)], idxd)
        pltpu.sync_copy(src2d_hbm.at[pl.ds(r, NSUB)], idxs)
        copies = []
        for j in range(NSUB):
            copies.append(pltpu.async_copy(
                ax_hbm.at[idxd.at[j]], adb.at[pl.ds(j * SUB, SUB)], sem))
            copies.append(pltpu.async_copy(
                bx_hbm.at[idxs.at[j]], bsb.at[pl.ds(j * SUB, SUB)], sem))
        for cp in copies:
            cp.wait()
        e0 = e0w + k * CHUNK
        pltpu.sync_copy(adb, ad_out.at[pl.ds(e0, CHUNK)])
        pltpu.sync_copy(bsb, bs_out.at[pl.ds(e0, CHUNK)])
        return carry

    lax.fori_loop(0, NCHUNK, chunk, 0)


@functools.partial(
    pl.kernel,
    mesh=_SC_MESH,
    out_type=jax.ShapeDtypeStruct((NC, N, ROWW), jnp.float32),
    scratch_types=[
        pltpu.VMEM((NSUB, SUB), jnp.int32),
        pltpu.VMEM((CHUNK, ROWW), jnp.float32),
        pltpu.VMEM_SHARED((N, ROWW), jnp.float32),
        pltpu.SemaphoreType.DMA,
    ],
)
def _sc_scatter(msgx_hbm, dst2d_hbm, zeros_hbm, out_hbm, idxb, mbuf, acc, sem):
    c = lax.axis_index("c")
    s = lax.axis_index("s")
    pltpu.sync_copy(zeros_hbm.at[pl.ds(s * ROWS_PT, ROWS_PT)],
                    acc.at[pl.ds(s * ROWS_PT, ROWS_PT)])
    plsc.subcore_barrier()
    wid = c * NS + s
    row0 = wid * IDXR_PW
    e0w = wid * EPW

    def chunk(k, carry):
        r = row0 + k * NSUB
        pltpu.sync_copy(dst2d_hbm.at[pl.ds(r, NSUB)], idxb)
        pltpu.sync_copy(msgx_hbm.at[pl.ds(e0w + k * CHUNK, CHUNK)], mbuf)
        for j in range(NSUB):
            pltpu.sync_copy(mbuf.at[pl.ds(j * SUB, SUB)],
                            acc.at[idxb.at[j]], add=True)
        return carry

    lax.fori_loop(0, NCHUNK, chunk, 0)
    plsc.subcore_barrier()
    pltpu.sync_copy(acc.at[pl.ds(s * ROWS_PT, ROWS_PT)],
                    out_hbm.at[c].at[pl.ds(s * ROWS_PT, ROWS_PT)])


def kernel(h, x, edge_index, mask_ligand, edge_attr, We1, be1, We2, be2,
           Winf, binf, Wx1, bx1, Wx2, Wn1, bn1, Wn2, bn2):
    xpad = jnp.pad(x, ((0, 0), (0, XW - 3)))
    dst2d = edge_index[1].reshape(IDXROWS, SUB)
    src2d = edge_index[0].reshape(IDXROWS, SUB)

    We1a = We1[:HID]
    We1b = We1[HID:2 * HID]
    We1d = We1[2 * HID:2 * HID + NUM_G]
    We1e = We1[2 * HID + NUM_G:]

    axt, bxt = _tc_pre(h, xpad, We1a, We1b)
    adx, bsx = _sc_gather(axt, bxt, dst2d, src2d)
    msgx = _tc_edge(adx, bsx, edge_attr, We1d, We1e, be1.reshape(1, HID),
                    We2, be2.reshape(1, HID), Winf.T, binf.reshape(1, 1),
                    Wx1, bx1.reshape(1, HID), Wx2.T)
    zeros = jnp.zeros((N, ROWW), jnp.float32)
    parts = _sc_scatter(msgx, dst2d, zeros)
    mask_f = mask_ligand.astype(jnp.float32).reshape(N, 1)
    h_out, xout_pad = _tc_node(h, xpad, parts, mask_f, Wn1[:HID], Wn1[HID:],
                               bn1.reshape(1, HID), Wn2, bn2.reshape(1, HID))
    return h_out, xout_pad[:, :3]


# speedup vs baseline: 3.8553x; 3.8553x over previous
import functools

import jax
import jax.numpy as jnp
import numpy as np
from jax import lax
from jax.experimental import pallas as pl
from jax.experimental.pallas import tpu as pltpu
from jax.experimental.pallas import tpu_sc as plsc

N = 10000
E = 320000
HID = 128
XW = 8
NUM_G = 16
LANES = 16

NC = 2
NS = 16
NW = NC * NS
EPW = E // NW
SUB = 80
NSUB = 5
CHUNK = SUB * NSUB
NCHUNK = EPW // CHUNK
NCHUNKS_ALL = E // CHUNK
S_SUB = 40
S_NSUB = 5
S_CHUNK = S_SUB * S_NSUB
S_NCHUNK = EPW // S_CHUNK
S_NCHUNKS_ALL = E // S_CHUNK
NACC = 10240
ROWS_PT = NACC // NS


def _pre_body(h_ref, wa_ref, wb_ref, a_ref, b_ref):
    h = h_ref[...]
    a_ref[...] = jnp.dot(h, wa_ref[...], preferred_element_type=jnp.float32)
    b_ref[...] = jnp.dot(h, wb_ref[...], preferred_element_type=jnp.float32)


def _tc_pre(h, We1a, We1b):
    return pl.pallas_call(
        _pre_body,
        out_shape=(jax.ShapeDtypeStruct((N, HID), jnp.float32),
                   jax.ShapeDtypeStruct((N, HID), jnp.float32)),
    )(h, We1a, We1b)


_EB = 2560
_EGRID = E // _EB
_G_STEP = float(np.float32(10.0) / np.float32(NUM_G - 1))
_G_COEFF = float(-0.5 / np.linspace(0.0, 10.0, NUM_G)[1] ** 2)


def _edge_body(ad_ref, bs_ref, rel_ref, ea_ref, wd_ref, wea_ref, be1_ref,
               we2_ref, be2_ref, winf_ref, binf_ref, wx1_ref, bx1_ref,
               wx2_ref, msg_ref, xmsg_ref):
    t1pre = ad_ref[...] + bs_ref[...]
    rel = rel_ref[...]
    d_sq = jnp.sum(rel * rel, axis=1, keepdims=True)
    dist = jnp.sqrt(d_sq + 1e-8)
    offs = (lax.broadcasted_iota(jnp.int32, (1, NUM_G), 1)
            .astype(jnp.float32) * _G_STEP)
    dfeat = jnp.exp(_G_COEFF * (dist - offs) ** 2)
    t1 = (t1pre
          + jnp.dot(dfeat, wd_ref[...], preferred_element_type=jnp.float32)
          + be1_ref[...])
    ea = ea_ref[...]
    wea = wea_ref[...]
    for k in range(4):
        t1 = t1 + ea[:, k:k + 1] * wea[k:k + 1, :]
    u = t1 * jax.nn.sigmoid(t1)
    m1 = jnp.dot(u, we2_ref[...], preferred_element_type=jnp.float32) + be2_ref[...]
    mij = m1 * jax.nn.sigmoid(m1)
    eij = jax.nn.sigmoid(
        jnp.sum(mij * winf_ref[...], axis=1, keepdims=True) + binf_ref[...])
    v1 = jnp.dot(mij, wx1_ref[...], preferred_element_type=jnp.float32) + bx1_ref[...]
    v = v1 * jax.nn.sigmoid(v1)
    xg = jnp.tanh(jnp.sum(v * wx2_ref[...], axis=1, keepdims=True))
    xmsg = rel * (xg / (dist + 1.0))
    msg_ref[...] = mij * eij
    xmsg_ref[...] = jnp.concatenate(
        [xmsg, jnp.zeros((xmsg.shape[0], HID - XW), jnp.float32)], axis=1)


def _tc_edge(ad, bs, rel, edge_attr, We1d, We1e, be1, We2, be2, winf_row,
             binf, Wx1, bx1, wx2_row):
    full = lambda shape: pl.BlockSpec(shape, lambda i: (0, 0))
    return pl.pallas_call(
        _edge_body,
        grid=(_EGRID,),
        in_specs=[
            pl.BlockSpec((_EB, HID), lambda i: (i, 0)),
            pl.BlockSpec((_EB, HID), lambda i: (i, 0)),
            pl.BlockSpec((_EB, XW), lambda i: (i, 0)),
            pl.BlockSpec((_EB, 4), lambda i: (i, 0)),
            full((NUM_G, HID)),
            full((4, HID)),
            full((1, HID)),
            full((HID, HID)),
            full((1, HID)),
            full((1, HID)),
            full((1, 1)),
            full((HID, HID)),
            full((1, HID)),
            full((1, HID)),
        ],
        out_specs=(pl.BlockSpec((_EB, HID), lambda i: (i, 0)),
                   pl.BlockSpec((_EB, HID), lambda i: (i, 0))),
        out_shape=(jax.ShapeDtypeStruct((E, HID), jnp.float32),
                   jax.ShapeDtypeStruct((E, HID), jnp.float32)),
    )(ad, bs, rel, edge_attr, We1d, We1e, be1, We2, be2, winf_row, binf,
      Wx1, bx1, wx2_row)


def _node_body(h_ref, xp_ref, pm_ref, pd_ref, mask_ref, wn1a_ref, wn1b_ref,
               bn1_ref, wn2_ref, bn2_ref, hout_ref, xout_ref):
    h = h_ref[...]
    mi = pm_ref[0][:N] + pm_ref[1][:N]
    dx = pd_ref[0][:N, :XW] + pd_ref[1][:N, :XW]
    t1 = (jnp.dot(mi, wn1a_ref[...], preferred_element_type=jnp.float32)
          + jnp.dot(h, wn1b_ref[...], preferred_element_type=jnp.float32)
          + bn1_ref[...])
    t = t1 * jax.nn.sigmoid(t1)
    hout_ref[...] = h + jnp.dot(t, wn2_ref[...],
                                preferred_element_type=jnp.float32) + bn2_ref[...]
    xout_ref[...] = xp_ref[...] + dx * mask_ref[...]


def _tc_node(h, xpad, parts_msg, parts_dx, mask_f, Wn1a, Wn1b, bn1, Wn2, bn2):
    return pl.pallas_call(
        _node_body,
        out_shape=(jax.ShapeDtypeStruct((N, HID), jnp.float32),
                   jax.ShapeDtypeStruct((N, XW), jnp.float32)),
    )(h, xpad, parts_msg, parts_dx, mask_f, Wn1a, Wn1b, bn1, Wn2, bn2)


@functools.cache
def _sc_gather_kernel():
    mesh = plsc.VectorSubcoreMesh(core_axis_name="c", subcore_axis_name="s")
    return functools.partial(
        pl.kernel,
        mesh=mesh,
        out_type=(jax.ShapeDtypeStruct((E, HID), jnp.float32),
                  jax.ShapeDtypeStruct((E, HID), jnp.float32)),
        scratch_types=[
            pltpu.VMEM((NSUB, SUB), jnp.int32),
            pltpu.VMEM((NSUB, SUB), jnp.int32),
            pltpu.VMEM((CHUNK, HID), jnp.float32),
            pltpu.VMEM((CHUNK, HID), jnp.float32),
            pltpu.SemaphoreType.DMA,
        ],
    )(_sc_gather_body)


def _sc_gather(a, b, dst3d, src3d):
    return _sc_gather_kernel()(a, b, dst3d, src3d)


def _sc_gather_body(a_hbm, b_hbm, dst3d_hbm, src3d_hbm, ad_out, bs_out,
                    idxd, idxs, adb, bsb, sem):
    c = lax.axis_index("c")
    s = lax.axis_index("s")
    wid = s * NC + c
    g0 = wid * NCHUNK
    e0w = wid * EPW

    def chunk(k, carry):
        pltpu.sync_copy(dst3d_hbm.at[g0 + k], idxd)
        pltpu.sync_copy(src3d_hbm.at[g0 + k], idxs)
        copies = []
        for j in range(NSUB):
            copies.append(pltpu.async_copy(
                a_hbm.at[idxd.at[j]], adb.at[pl.ds(j * SUB, SUB)], sem))
            copies.append(pltpu.async_copy(
                b_hbm.at[idxs.at[j]], bsb.at[pl.ds(j * SUB, SUB)], sem))
        for cp in copies:
            cp.wait()
        e0 = e0w + k * CHUNK
        pltpu.sync_copy(adb, ad_out.at[pl.ds(e0, CHUNK)])
        pltpu.sync_copy(bsb, bs_out.at[pl.ds(e0, CHUNK)])
        return carry

    lax.fori_loop(0, NCHUNK, chunk, 0)


@functools.cache
def _sc_rel_kernel():
    mesh = plsc.VectorSubcoreMesh(core_axis_name="c", subcore_axis_name="s")
    return functools.partial(
        pl.kernel,
        mesh=mesh,
        compiler_params=pltpu.CompilerParams(needs_layout_passes=False),
        out_type=jax.ShapeDtypeStruct((E * XW,), jnp.float32),
        scratch_types=[
            pltpu.VMEM((N,), jnp.float32),
            pltpu.VMEM((N,), jnp.float32),
            pltpu.VMEM((N,), jnp.float32),
            pltpu.VMEM((NSUB, SUB), jnp.int32),
            pltpu.VMEM((NSUB, SUB), jnp.int32),
            pltpu.VMEM((CHUNK * XW,), jnp.float32),
            pltpu.SemaphoreType.DMA,
        ],
    )(_sc_rel_body)


def _sc_rel(x0, x1, x2, dst3d, src3d):
    return _sc_rel_kernel()(x0, x1, x2, dst3d, src3d)


def _sc_rel_body(x0_hbm, x1_hbm, x2_hbm, dst3d_hbm, src3d_hbm, rel_out,
                 x0b, x1b, x2b, idxd, idxs, relb, sem):
    c = lax.axis_index("c")
    s = lax.axis_index("s")
    wid = s * NC + c
    g0 = wid * NCHUNK
    e0w = wid * EPW
    pltpu.sync_copy(x0_hbm, x0b)
    pltpu.sync_copy(x1_hbm, x1b)
    pltpu.sync_copy(x2_hbm, x2b)

    def zero(v, carry):
        relb[pl.ds(v * LANES, LANES)] = jnp.zeros((LANES,), jnp.float32)
        return carry

    lax.fori_loop(0, CHUNK * XW // LANES, zero, 0)

    def chunk(k, carry):
        pltpu.sync_copy(dst3d_hbm.at[g0 + k], idxd)
        pltpu.sync_copy(src3d_hbm.at[g0 + k], idxs)
        for j in range(NSUB):
            for i in range(SUB // LANES):
                ivd = idxd[j, pl.ds(i * LANES, LANES)]
                ivs = idxs[j, pl.ds(i * LANES, LANES)]
                base = (j * SUB + i * LANES) * XW
                flat = lax.iota(jnp.int32, LANES) * XW + base
                for comp, xb in ((0, x0b), (1, x1b), (2, x2b)):
                    d = plsc.load_gather(xb, [ivd])
                    sv = plsc.load_gather(xb, [ivs])
                    plsc.store_scatter(relb, [flat + comp], d - sv)
        pltpu.sync_copy(relb, rel_out.at[pl.ds((e0w + k * CHUNK) * XW,
                                               CHUNK * XW)])
        return carry

    lax.fori_loop(0, NCHUNK, chunk, 0)


@functools.cache
def _sc_scatter_kernel():
    mesh = plsc.VectorSubcoreMesh(core_axis_name="c", subcore_axis_name="s")
    return functools.partial(
        pl.kernel,
        mesh=mesh,
        out_type=(jax.ShapeDtypeStruct((NC, NACC, HID), jnp.float32),
                  jax.ShapeDtypeStruct((NC, NACC, HID), jnp.float32)),
        scratch_types=[
            pltpu.VMEM((S_NSUB, S_SUB), jnp.int32),
            pltpu.VMEM((S_CHUNK, HID), jnp.float32),
            pltpu.VMEM_SHARED((NACC, HID), jnp.float32),
            pltpu.SemaphoreType.DMA,
        ],
    )(_sc_scatter_body)


def _sc_scatter(msg, xmsg, dst3d, zeros):
    return _sc_scatter_kernel()(msg, xmsg, dst3d, zeros)


def _sc_scatter_body(msg_hbm, xmsg_hbm, dst3d_hbm, zeros_hbm, pm_hbm, pd_hbm,
                     idxb, mbuf, acc, sem):
    c = lax.axis_index("c")
    s = lax.axis_index("s")
    wid = c * NS + s
    g0 = wid * S_NCHUNK
    e0w = wid * EPW
    rows = pl.ds(s * ROWS_PT, ROWS_PT)

    for src_hbm, out_hbm in ((msg_hbm, pm_hbm), (xmsg_hbm, pd_hbm)):
        pltpu.sync_copy(zeros_hbm.at[rows], acc.at[rows])
        plsc.subcore_barrier()

        def chunk(k, carry):
            pltpu.sync_copy(dst3d_hbm.at[g0 + k], idxb)
            pltpu.sync_copy(src_hbm.at[pl.ds(e0w + k * S_CHUNK, S_CHUNK)], mbuf)
            for j in range(S_NSUB):
                pltpu.sync_copy(mbuf.at[pl.ds(j * S_SUB, S_SUB)],
                                acc.at[idxb.at[j]], add=True)
            return carry

        lax.fori_loop(0, S_NCHUNK, chunk, 0)
        plsc.subcore_barrier()
        pltpu.sync_copy(acc.at[rows], out_hbm.at[c].at[rows])
        plsc.subcore_barrier()


def kernel(h, x, edge_index, mask_ligand, edge_attr, We1, be1, We2, be2,
           Winf, binf, Wx1, bx1, Wx2, Wn1, bn1, Wn2, bn2):
    xpad = jnp.pad(x, ((0, 0), (0, XW - 3)))
    dst3d = edge_index[1].reshape(NCHUNKS_ALL, NSUB, SUB)
    src3d = edge_index[0].reshape(NCHUNKS_ALL, NSUB, SUB)

    We1a = We1[:HID]
    We1b = We1[HID:2 * HID]
    We1d = We1[2 * HID:2 * HID + NUM_G]
    We1e = We1[2 * HID + NUM_G:]

    a, b = _tc_pre(h, We1a, We1b)
    ad, bs = _sc_gather(a, b, dst3d, src3d)
    rel = _sc_rel(x[:, 0], x[:, 1], x[:, 2], dst3d, src3d).reshape(E, XW)
    msg, xmsg = _tc_edge(ad, bs, rel, edge_attr, We1d, We1e,
                         be1.reshape(1, HID), We2, be2.reshape(1, HID),
                         Winf.T, binf.reshape(1, 1), Wx1,
                         bx1.reshape(1, HID), Wx2.T)
    zeros = jnp.zeros((NACC, HID), jnp.float32)
    dst3d_s = edge_index[1].reshape(S_NCHUNKS_ALL, S_NSUB, S_SUB)
    parts_msg, parts_dx = _sc_scatter(msg, xmsg, dst3d_s, zeros)
    mask_f = mask_ligand.astype(jnp.float32).reshape(N, 1)
    h_out, xout_pad = _tc_node(h, xpad, parts_msg, parts_dx, mask_f,
                               Wn1[:HID], Wn1[HID:], bn1.reshape(1, HID),
                               Wn2, bn2.reshape(1, HID))
    return h_out, xout_pad[:, :3]
